# baseline (speedup 1.0000x reference)
import functools

import jax
import jax.numpy as jnp
from jax import lax
from jax.experimental import pallas as pl
from jax.experimental.pallas import tpu as pltpu
from jax.experimental.pallas import tpu_sc as plsc

B = 4096
L = 200
EMBED = 32

NUM_CORES = 2
NUM_SUBCORES = 16
NW = NUM_CORES * NUM_SUBCORES
BW = B // NW
LC = 5
NCHUNK = L // LC
CROWS = LC * BW
PER_W = L * BW


def _body(x_hbm, table_hbm, pos_hbm, out_hbm,
          idx_all, r0, r1, t0, t1, pos_v,
          gsem0, gsem1, osem0, osem1):
    wid = lax.axis_index("s") * NUM_CORES + lax.axis_index("c")

    pltpu.sync_copy(pos_hbm, pos_v)
    pltpu.sync_copy(x_hbm.at[pl.ds(wid * PER_W, PER_W)], idx_all)

    iota = lax.iota(jnp.int32, 16)
    rowvs = [[iota + (j * BW + bg * 16) for bg in range(8)] for j in range(LC)]

    rbufs = (r0, r1)
    tbufs = (t0, t1)
    gsems = (gsem0, gsem1)
    osems = (osem0, osem1)

    pltpu.async_copy(table_hbm.at[idx_all.at[pl.ds(0, CROWS)]], r0, gsem0)
    pltpu.async_copy(table_hbm.at[idx_all.at[pl.ds(CROWS, CROWS)]], r1, gsem1)

    def step(i, _):
        for par in (0, 1):
            c = 2 * i + par
            rv, tv = rbufs[par], tbufs[par]
            gsem, osem = gsems[par], osems[par]
            lbase = c * LC

            pltpu.make_async_copy(
                table_hbm.at[idx_all.at[pl.ds(0, CROWS)]], rv, gsem).wait()
            @pl.when(i >= 1)
            def _():
                pltpu.make_async_copy(
                    tv, out_hbm.at[pl.ds(0, LC), :, wid], osem).wait()

            def erow(e, _):
                colv = jnp.full((16,), e, jnp.int32)
                eh = e >> 3
                elo = e & 7
                for j in range(LC):
                    l = lbase + j
                    ps = plsc.load_gather(
                        pos_v, [jnp.full((16,), l, jnp.int32), colv])
                    for bg in range(8):
                        v = plsc.load_gather(rv, [rowvs[j][bg], colv]) + ps
                        tv[j, eh, elo, pl.ds(bg * 16, 16)] = v
                return 0

            lax.fori_loop(0, EMBED, erow, 0)

            pltpu.async_copy(tv, out_hbm.at[pl.ds(lbase, LC), :, wid], osem)

            @pl.when(i < (NCHUNK // 2) - 1)
            def _():
                pltpu.async_copy(
                    table_hbm.at[idx_all.at[pl.ds((c + 2) * CROWS, CROWS)]],
                    rv, gsem)
        return 0

    lax.fori_loop(0, NCHUNK // 2, step, 0)

    pltpu.make_async_copy(t0, out_hbm.at[pl.ds(0, LC), :, wid], osem0).wait()
    pltpu.make_async_copy(t1, out_hbm.at[pl.ds(0, LC), :, wid], osem1).wait()


@jax.jit
def _embed(x_w, table, pos_table):
    mesh = plsc.VectorSubcoreMesh(
        core_axis_name="c", subcore_axis_name="s",
        num_cores=NUM_CORES, num_subcores=NUM_SUBCORES,
    )
    run = functools.partial(
        pl.kernel,
        out_type=jax.ShapeDtypeStruct((L, EMBED // 8, NW, 8, BW), jnp.float32),
        mesh=mesh,
        scratch_types=[
            pltpu.VMEM((PER_W,), jnp.int32),
            pltpu.VMEM((CROWS, EMBED), jnp.float32),
            pltpu.VMEM((CROWS, EMBED), jnp.float32),
            pltpu.VMEM((LC, EMBED // 8, 8, BW), jnp.float32),
            pltpu.VMEM((LC, EMBED // 8, 8, BW), jnp.float32),
            pltpu.VMEM((L, EMBED), jnp.float32),
            pltpu.SemaphoreType.DMA,
            pltpu.SemaphoreType.DMA,
            pltpu.SemaphoreType.DMA,
            pltpu.SemaphoreType.DMA,
        ],
        compiler_params=pltpu.CompilerParams(
            use_tc_tiling_on_sc=False, needs_layout_passes=False),
    )(_body)
    return run(x_w, table, pos_table)


def kernel(x, table, pos_table):
    x_w = (x.T.astype(jnp.int32)
           .reshape(L, NW, BW).transpose(1, 0, 2).reshape(-1))
    arr = _embed(x_w, table, pos_table)
    return arr.transpose(2, 4, 0, 1, 3).reshape(B, L, EMBED)


# speedup vs baseline: 1.2810x; 1.2810x over previous
import functools

import jax
import jax.numpy as jnp
from jax import lax
from jax.experimental import pallas as pl
from jax.experimental.pallas import tpu as pltpu
from jax.experimental.pallas import tpu_sc as plsc

B = 4096
L = 200
EMBED = 32
EP = EMBED + 1

NUM_CORES = 2
NUM_SUBCORES = 16
NW = NUM_CORES * NUM_SUBCORES
BW = B // NW
LC = 4
NCHUNK = L // LC
CROWS = LC * BW
PER_W = L * BW


def _body(x_hbm, table_hbm, pos_hbm, out_hbm,
          idx_all, r0, r1, rs, t0, t1, pos_v,
          gsem0, gsem1, osem0, osem1):
    wid = lax.axis_index("s") * NUM_CORES + lax.axis_index("c")

    pltpu.sync_copy(pos_hbm, pos_v)
    pltpu.sync_copy(x_hbm.at[pl.ds(wid * PER_W, PER_W)], idx_all)

    iota = lax.iota(jnp.int32, 16)
    rowvs = [[(iota + (j * BW + bg * 16)) * EP for bg in range(8)]
             for j in range(LC)]

    rbufs = (r0, r1)
    tbufs = (t0, t1)
    gsems = (gsem0, gsem1)
    osems = (osem0, osem1)

    pltpu.async_copy(table_hbm.at[idx_all.at[pl.ds(0, CROWS)]], r0, gsem0)
    pltpu.async_copy(table_hbm.at[idx_all.at[pl.ds(CROWS, CROWS)]], r1, gsem1)

    def step(i, _):
        for par in (0, 1):
            c = 2 * i + par
            rv, tv = rbufs[par], tbufs[par]
            gsem, osem = gsems[par], osems[par]
            lbase = c * LC

            pltpu.make_async_copy(
                table_hbm.at[idx_all.at[pl.ds(0, CROWS)]], rv, gsem).wait()
            @pl.when(i >= 1)
            def _():
                pltpu.make_async_copy(
                    tv, out_hbm.at[pl.ds(0, LC), :, wid], osem).wait()

            def repack(g, _):
                b0 = g * 8
                for k in range(8):
                    b = b0 + k
                    rs[pl.ds(b * EP, 16)] = rv[b, pl.ds(0, 16)]
                    rs[pl.ds(b * EP + 16, 16)] = rv[b, pl.ds(16, 16)]
                return 0

            lax.fori_loop(0, CROWS // 8, repack, 0)

            def erow(e, _):
                colv = jnp.full((16,), e, jnp.int32)
                eh = e >> 3
                elo = e & 7
                for j in range(LC):
                    l = lbase + j
                    ps = plsc.load_gather(
                        pos_v, [jnp.full((16,), l, jnp.int32), colv])
                    for bg in range(8):
                        v = plsc.load_gather(rs, [rowvs[j][bg] + colv]) + ps
                        tv[j, eh, elo, pl.ds(bg * 16, 16)] = v
                return 0

            lax.fori_loop(0, EMBED, erow, 0)

            pltpu.async_copy(tv, out_hbm.at[pl.ds(lbase, LC), :, wid], osem)

            @pl.when(i < (NCHUNK // 2) - 1)
            def _():
                pltpu.async_copy(
                    table_hbm.at[idx_all.at[pl.ds((c + 2) * CROWS, CROWS)]],
                    rv, gsem)
        return 0

    lax.fori_loop(0, NCHUNK // 2, step, 0)

    pltpu.make_async_copy(t0, out_hbm.at[pl.ds(0, LC), :, wid], osem0).wait()
    pltpu.make_async_copy(t1, out_hbm.at[pl.ds(0, LC), :, wid], osem1).wait()


@jax.jit
def _embed(x_w, table, pos_table):
    mesh = plsc.VectorSubcoreMesh(
        core_axis_name="c", subcore_axis_name="s",
        num_cores=NUM_CORES, num_subcores=NUM_SUBCORES,
    )
    run = functools.partial(
        pl.kernel,
        out_type=jax.ShapeDtypeStruct((L, EMBED // 8, NW, 8, BW), jnp.float32),
        mesh=mesh,
        scratch_types=[
            pltpu.VMEM((PER_W,), jnp.int32),
            pltpu.VMEM((CROWS, EMBED), jnp.float32),
            pltpu.VMEM((CROWS, EMBED), jnp.float32),
            pltpu.VMEM((CROWS * EP,), jnp.float32),
            pltpu.VMEM((LC, EMBED // 8, 8, BW), jnp.float32),
            pltpu.VMEM((LC, EMBED // 8, 8, BW), jnp.float32),
            pltpu.VMEM((L, EMBED), jnp.float32),
            pltpu.SemaphoreType.DMA,
            pltpu.SemaphoreType.DMA,
            pltpu.SemaphoreType.DMA,
            pltpu.SemaphoreType.DMA,
        ],
        compiler_params=pltpu.CompilerParams(
            use_tc_tiling_on_sc=False, needs_layout_passes=False),
    )(_body)
    return run(x_w, table, pos_table)


def kernel(x, table, pos_table):
    x_w = (x.T.astype(jnp.int32)
           .reshape(L, NW, BW).transpose(1, 0, 2).reshape(-1))
    arr = _embed(x_w, table, pos_table)
    return arr.transpose(2, 4, 0, 1, 3).reshape(B, L, EMBED)


# speedup vs baseline: 1.8834x; 1.4703x over previous
import functools

import jax
import jax.numpy as jnp
from jax import lax
from jax.experimental import pallas as pl
from jax.experimental.pallas import tpu as pltpu
from jax.experimental.pallas import tpu_sc as plsc

B = 4096
L = 200
EMBED = 32
EP = EMBED + 1

NUM_CORES = 2
NUM_SUBCORES = 16
NW = NUM_CORES * NUM_SUBCORES
BW = B // NW
LC = 4
NCHUNK = L // LC
CROWS = LC * BW
PER_W = L * BW


def _body(x_hbm, table_hbm, pos_hbm, out_hbm,
          idx_all, r0, r1, rs, t0, t1, pos_v,
          gsem0, gsem1, osem0, osem1):
    wid = lax.axis_index("s") * NUM_CORES + lax.axis_index("c")

    pltpu.sync_copy(pos_hbm, pos_v)
    pltpu.sync_copy(x_hbm.at[pl.ds(wid * PER_W, PER_W)], idx_all)

    iota = lax.iota(jnp.int32, 16)
    rowvs = [[(iota + (j * BW + bg * 16)) * EP for bg in range(8)]
             for j in range(LC)]

    rbufs = (r0, r1)
    tbufs = (t0, t1)
    gsems = (gsem0, gsem1)
    osems = (osem0, osem1)

    pltpu.async_copy(table_hbm.at[idx_all.at[pl.ds(0, CROWS)]], r0, gsem0)
    pltpu.async_copy(table_hbm.at[idx_all.at[pl.ds(CROWS, CROWS)]], r1, gsem1)

    def step(i, _):
        for par in (0, 1):
            c = 2 * i + par
            rv, tv = rbufs[par], tbufs[par]
            gsem, osem = gsems[par], osems[par]
            lbase = c * LC

            pltpu.make_async_copy(
                table_hbm.at[idx_all.at[pl.ds(0, CROWS)]], rv, gsem).wait()
            @pl.when(i >= 1)
            def _():
                pltpu.make_async_copy(
                    tv, out_hbm.at[pl.ds(0, LC), :, wid], osem).wait()

            for j in range(LC):
                l = lbase + j
                p_lo = pos_v[l, pl.ds(0, 16)]
                p_hi = pos_v[l, pl.ds(16, 16)]

                @plsc.parallel_loop(0, BW // 8, unroll=2)
                def repack(g, j=j, p_lo=p_lo, p_hi=p_hi):
                    b0 = j * BW + g * 8
                    for k in range(8):
                        b = b0 + k
                        rs[pl.ds(b * EP, 16)] = rv[b, pl.ds(0, 16)] + p_lo
                        rs[pl.ds(b * EP + 16, 16)] = rv[b, pl.ds(16, 16)] + p_hi

            @plsc.parallel_loop(0, EMBED, unroll=2)
            def erow(e):
                colv = jnp.full((16,), e, jnp.int32)
                eh = e >> 3
                elo = e & 7
                for j in range(LC):
                    for bg in range(8):
                        v = plsc.load_gather(rs, [rowvs[j][bg] + colv])
                        tv[j, eh, elo, pl.ds(bg * 16, 16)] = v

            pltpu.async_copy(tv, out_hbm.at[pl.ds(lbase, LC), :, wid], osem)

            @pl.when(i < (NCHUNK // 2) - 1)
            def _():
                pltpu.async_copy(
                    table_hbm.at[idx_all.at[pl.ds((c + 2) * CROWS, CROWS)]],
                    rv, gsem)
        return 0

    lax.fori_loop(0, NCHUNK // 2, step, 0)

    pltpu.make_async_copy(t0, out_hbm.at[pl.ds(0, LC), :, wid], osem0).wait()
    pltpu.make_async_copy(t1, out_hbm.at[pl.ds(0, LC), :, wid], osem1).wait()


@jax.jit
def _embed(x_w, table, pos_table):
    mesh = plsc.VectorSubcoreMesh(
        core_axis_name="c", subcore_axis_name="s",
        num_cores=NUM_CORES, num_subcores=NUM_SUBCORES,
    )
    run = functools.partial(
        pl.kernel,
        out_type=jax.ShapeDtypeStruct((L, EMBED // 8, NW, 8, BW), jnp.float32),
        mesh=mesh,
        scratch_types=[
            pltpu.VMEM((PER_W,), jnp.int32),
            pltpu.VMEM((CROWS, EMBED), jnp.float32),
            pltpu.VMEM((CROWS, EMBED), jnp.float32),
            pltpu.VMEM((CROWS * EP,), jnp.float32),
            pltpu.VMEM((LC, EMBED // 8, 8, BW), jnp.float32),
            pltpu.VMEM((LC, EMBED // 8, 8, BW), jnp.float32),
            pltpu.VMEM((L, EMBED), jnp.float32),
            pltpu.SemaphoreType.DMA,
            pltpu.SemaphoreType.DMA,
            pltpu.SemaphoreType.DMA,
            pltpu.SemaphoreType.DMA,
        ],
        compiler_params=pltpu.CompilerParams(
            use_tc_tiling_on_sc=False, needs_layout_passes=False),
    )(_body)
    return run(x_w, table, pos_table)


def kernel(x, table, pos_table):
    x_w = (x.T.astype(jnp.int32)
           .reshape(L, NW, BW).transpose(1, 0, 2).reshape(-1))
    arr = _embed(x_w, table, pos_table)
    return arr.transpose(2, 4, 0, 1, 3).reshape(B, L, EMBED)
